# Initial kernel scaffold; baseline (speedup 1.0000x reference)
import jax
import jax.numpy as jnp
from jax.experimental import pallas as pl


def kernel(h, x, u, state_pos, action_pos, a2s_edge_index, a2s_dis, s2s_edge_index, s2s_dis, W_u2h, b_u2h, W_x2h, b_x2h, W_upd, b_upd):
    raise NotImplementedError("write your pallas kernel here")


# speedup vs baseline: 10.6113x; 10.6113x over previous
import jax
import jax.numpy as jnp
from jax import lax
from jax.experimental import pallas as pl
from jax.experimental.pallas import tpu as pltpu
from jax.experimental.pallas import tpu_sc as plsc

N = 10000
HID = 64
E = 320000
NC, NS, L = 2, 16, 16
CH = 128
CHUNKS = 160
E_PAD = NS * CH * CHUNKS
ROWS_PER_TILE = 640
N_PAD = NS * ROWS_PER_TILE
EROWS = E_PAD // CH


BLK = 2000


def _mm_body(a, w, out):
    out[:] = jnp.dot(a[:], w[:], preferred_element_type=jnp.float32)


def _row_matmul(a, w):
    n, k = a.shape
    return pl.pallas_call(
        _mm_body,
        grid=(n // BLK,),
        in_specs=[
            pl.BlockSpec((BLK, k), lambda i: (i, 0)),
            pl.BlockSpec((k, HID), lambda i: (0, 0)),
        ],
        out_specs=pl.BlockSpec((BLK, HID), lambda i: (i, 0)),
        out_shape=jax.ShapeDtypeStruct((n, HID), jnp.float32),
    )(a, w)


def _seg_body(tbl, src, dst, dis, seg_out, cnt_out, dis_out,
              src_v, dst_v, dis_v, ones_v, rows_v, acc_seg, acc_cnt, acc_dis):
    c = lax.axis_index("c")
    s = lax.axis_index("s")

    ones16 = jnp.ones((L,), jnp.float32)
    zeros16 = jnp.zeros((L,), jnp.float32)

    def _init_ones(i, _):
        ones_v[pl.ds(i * L, L)] = ones16
        return 0
    lax.fori_loop(0, CH // L, _init_ones, 0)

    def _zero_rows(i, _):
        rows_v[i, pl.ds(0, L)] = zeros16
        rows_v[i, pl.ds(L, L)] = zeros16
        rows_v[i, pl.ds(2 * L, L)] = zeros16
        rows_v[i, pl.ds(3 * L, L)] = zeros16
        return 0
    lax.fori_loop(0, CH, _zero_rows, 0)

    r0 = s * ROWS_PER_TILE
    for i in range(ROWS_PER_TILE // CH):
        pltpu.sync_copy(rows_v, acc_seg.at[pl.ds(r0 + i * CH, CH)])
    def _zero_vec(i, _):
        dis_v[0, pl.ds(i * L, L)] = zeros16
        return 0
    lax.fori_loop(0, CH // L, _zero_vec, 0)
    for i in range(ROWS_PER_TILE // CH):
        pltpu.sync_copy(dis_v.at[0], acc_cnt.at[pl.ds(r0 + i * CH, CH)])
        pltpu.sync_copy(dis_v.at[0], acc_dis.at[pl.ds(r0 + i * CH, CH)])

    row0 = c * EROWS + s * CHUNKS
    pltpu.sync_copy(src.at[pl.ds(row0, CHUNKS)], src_v)
    pltpu.sync_copy(dst.at[pl.ds(row0, CHUNKS)], dst_v)
    pltpu.sync_copy(dis.at[pl.ds(row0, CHUNKS)], dis_v)

    plsc.subcore_barrier()

    def _chunk(j, _):
        pltpu.sync_copy(tbl.at[src_v.at[j]], rows_v)
        pltpu.sync_copy(rows_v, acc_seg.at[dst_v.at[j]], add=True)
        pltpu.sync_copy(dis_v.at[j], acc_dis.at[dst_v.at[j]], add=True)
        pltpu.sync_copy(ones_v, acc_cnt.at[dst_v.at[j]], add=True)
        return 0
    lax.fori_loop(0, CHUNKS, _chunk, 0)

    plsc.subcore_barrier()

    o0 = c * N_PAD + r0
    pltpu.sync_copy(acc_seg.at[pl.ds(r0, ROWS_PER_TILE)],
                    seg_out.at[pl.ds(o0, ROWS_PER_TILE)])
    pltpu.sync_copy(acc_cnt.at[pl.ds(r0, ROWS_PER_TILE)],
                    cnt_out.at[pl.ds(o0, ROWS_PER_TILE)])
    pltpu.sync_copy(acc_dis.at[pl.ds(r0, ROWS_PER_TILE)],
                    dis_out.at[pl.ds(o0, ROWS_PER_TILE)])


def _segment_sums(tbl, src2d, dst2d, dis2d):
    mesh = plsc.VectorSubcoreMesh(core_axis_name="c", subcore_axis_name="s",
                                  num_cores=NC, num_subcores=NS)
    f32 = jnp.float32
    return pl.kernel(
        _seg_body,
        out_type=[
            jax.ShapeDtypeStruct((NC * N_PAD, HID), f32),
            jax.ShapeDtypeStruct((NC * N_PAD,), f32),
            jax.ShapeDtypeStruct((NC * N_PAD,), f32),
        ],
        mesh=mesh,
        scratch_types=[
            pltpu.VMEM((CHUNKS, CH), jnp.int32),
            pltpu.VMEM((CHUNKS, CH), jnp.int32),
            pltpu.VMEM((CHUNKS, CH), f32),
            pltpu.VMEM((CH,), f32),
            pltpu.VMEM((CH, HID), f32),
            pltpu.VMEM_SHARED((N_PAD, HID), f32),
            pltpu.VMEM_SHARED((N_PAD,), f32),
            pltpu.VMEM_SHARED((N_PAD,), f32),
        ],
        compiler_params=pltpu.CompilerParams(use_tc_tiling_on_sc=False),
    )(tbl, src2d, dst2d, dis2d)


def _final_body(pos, h, x, seg_a, seg_s, cns,
                wsp_a, bu, wdis_a, wsp_s, bx, wdis_s,
                w1, w2, w3, w4, w5, bd, out):
    dot = lambda a, b: jnp.dot(a[:], b[:], preferred_element_type=jnp.float32)
    cnt_a = cns[:, 0:1]
    sd_a = cns[:, 1:2]
    cnt_s = cns[:, 2:3]
    sd_s = cns[:, 3:4]
    su = seg_a[:] + cnt_a * (dot(pos, wsp_a) + bu[:]) + sd_a * wdis_a[:]
    mx = seg_s[:] + cnt_s * (dot(pos, wsp_s) + bx[:]) + sd_s * wdis_s[:]
    mx = mx / jnp.maximum(cnt_s, 1.0)
    out[:] = (dot(pos, w1) + dot(h, w2) + dot(su, w3) + dot(mx, w4)
              + dot(x, w5) + bd[:])


def _final(pos, h, x, seg_a, seg_s, cns,
           wsp_a, bu, wdis_a, wsp_s, bx, wdis_s, w1, w2, w3, w4, w5, bd):
    row = lambda k: pl.BlockSpec((BLK, k), lambda i: (i, 0))
    full = lambda a: pl.BlockSpec(a.shape, lambda i: (0, 0))
    return pl.pallas_call(
        _final_body,
        grid=(N // BLK,),
        in_specs=[row(2), row(HID), row(128), row(HID), row(HID), row(4),
                  full(wsp_a), full(bu), full(wdis_a),
                  full(wsp_s), full(bx), full(wdis_s),
                  full(w1), full(w2), full(w3), full(w4), full(w5), full(bd)],
        out_specs=pl.BlockSpec((BLK, HID), lambda i: (i, 0)),
        out_shape=jax.ShapeDtypeStruct((N, HID), jnp.float32),
    )(pos, h, x, seg_a, seg_s, cns,
      wsp_a, bu, wdis_a, wsp_s, bx, wdis_s, w1, w2, w3, w4, w5, bd)


def _pad_edges(a, fill):
    return jnp.concatenate([a, jnp.full((E_PAD - E,), fill, a.dtype)])


def kernel(h, x, u, state_pos, action_pos, a2s_edge_index, a2s_dis,
           s2s_edge_index, s2s_dis, W_u2h, b_u2h, W_x2h, b_x2h, W_upd, b_upd):
    f32 = jnp.float32

    a_in = jnp.concatenate([u, action_pos], axis=1)
    s_in = jnp.concatenate([x, h, state_pos], axis=1)
    tbl_a = _row_matmul(a_in, W_u2h[:, :18].T)
    tbl_s = _row_matmul(s_in, W_x2h[:, :194].T)
    tbl = jnp.concatenate([tbl_a, tbl_s], axis=0)

    src2d = jnp.concatenate([
        _pad_edges(a2s_edge_index[0], 0),
        _pad_edges(s2s_edge_index[0], 0) + N,
    ]).reshape(2 * EROWS, CH)
    dst2d = jnp.concatenate([
        _pad_edges(a2s_edge_index[1], N),
        _pad_edges(s2s_edge_index[1], N),
    ]).reshape(2 * EROWS, CH)
    dis2d = jnp.concatenate([
        _pad_edges(a2s_dis[:, 0], 0.0),
        _pad_edges(s2s_dis[:, 0], 0.0),
    ]).reshape(2 * EROWS, CH)

    seg, cnt, sdis = _segment_sums(tbl, src2d, dst2d, dis2d)

    seg_a, seg_s = seg[:N], seg[N_PAD:N_PAD + N]
    cns = jnp.stack([cnt[:N], sdis[:N], cnt[N_PAD:N_PAD + N],
                     sdis[N_PAD:N_PAD + N]], axis=1)

    out = _final(
        state_pos, h, x, seg_a, seg_s, cns,
        W_u2h[:, 18:20].T, b_u2h[None, :], W_u2h[:, 20][None, :],
        W_x2h[:, 194:196].T, b_x2h[None, :], W_x2h[:, 196][None, :],
        W_upd[:, 0:2].T, W_upd[:, 2:66].T, W_upd[:, 66:130].T,
        W_upd[:, 130:194].T, W_upd[:, 194:322].T, b_upd[None, :])
    return out


# speedup vs baseline: 12.2887x; 1.1581x over previous
import jax
import jax.numpy as jnp
from jax import lax
from jax.experimental import pallas as pl
from jax.experimental.pallas import tpu as pltpu
from jax.experimental.pallas import tpu_sc as plsc

N = 10000
HID = 64
E = 320000
NC, NS, L = 2, 16, 16
CH = 128
CHUNKS = 160
E_PAD = NS * CH * CHUNKS
ROWS_PER_TILE = 640
N_PAD = NS * ROWS_PER_TILE
EROWS = E_PAD // CH


BLK = 2000


def _mm_body(a, w, out):
    out[:] = jnp.dot(a[:], w[:], preferred_element_type=jnp.float32)


def _row_matmul(a, w):
    n, k = a.shape
    return pl.pallas_call(
        _mm_body,
        grid=(n // BLK,),
        in_specs=[
            pl.BlockSpec((BLK, k), lambda i: (i, 0)),
            pl.BlockSpec((k, HID), lambda i: (0, 0)),
        ],
        out_specs=pl.BlockSpec((BLK, HID), lambda i: (i, 0)),
        out_shape=jax.ShapeDtypeStruct((n, HID), jnp.float32),
    )(a, w)


def _seg_body(tbl, src, dst, dis, seg_out, cnt_out, dis_out,
              src_v, dst_v, dis_v, ones_v, rows_v, rows_w,
              acc_seg, acc_cnt, acc_dis, gsem, ssem):
    c = lax.axis_index("c")
    s = lax.axis_index("s")

    ones16 = jnp.ones((L,), jnp.float32)
    zeros16 = jnp.zeros((L,), jnp.float32)

    def _init_ones(i, _):
        ones_v[pl.ds(i * L, L)] = ones16
        return 0
    lax.fori_loop(0, CH // L, _init_ones, 0)

    def _zero_rows(i, _):
        rows_v[i, pl.ds(0, L)] = zeros16
        rows_v[i, pl.ds(L, L)] = zeros16
        rows_v[i, pl.ds(2 * L, L)] = zeros16
        rows_v[i, pl.ds(3 * L, L)] = zeros16
        return 0
    lax.fori_loop(0, CH, _zero_rows, 0)

    r0 = s * ROWS_PER_TILE
    for i in range(ROWS_PER_TILE // CH):
        pltpu.sync_copy(rows_v, acc_seg.at[pl.ds(r0 + i * CH, CH)])
    def _zero_vec(i, _):
        dis_v[0, pl.ds(i * L, L)] = zeros16
        return 0
    lax.fori_loop(0, CH // L, _zero_vec, 0)
    for i in range(ROWS_PER_TILE // CH):
        pltpu.sync_copy(dis_v.at[0], acc_cnt.at[pl.ds(r0 + i * CH, CH)])
        pltpu.sync_copy(dis_v.at[0], acc_dis.at[pl.ds(r0 + i * CH, CH)])

    row0 = c * EROWS + s * CHUNKS
    pltpu.sync_copy(src.at[pl.ds(row0, CHUNKS)], src_v)
    pltpu.sync_copy(dst.at[pl.ds(row0, CHUNKS)], dst_v)
    pltpu.sync_copy(dis.at[pl.ds(row0, CHUNKS)], dis_v)

    plsc.subcore_barrier()

    bufs = (rows_v, rows_w)

    def _gather_start(j, buf):
        pltpu.async_copy(tbl.at[src_v.at[j]], buf, gsem)

    def _gather_wait(j, buf):
        pltpu.make_async_copy(tbl.at[src_v.at[j]], buf, gsem).wait()

    def _scatter(j, buf):
        pltpu.async_copy(buf, acc_seg.at[dst_v.at[j]], ssem, add=True)
        pltpu.async_copy(dis_v.at[j], acc_dis.at[dst_v.at[j]], ssem, add=True)
        pltpu.async_copy(ones_v, acc_cnt.at[dst_v.at[j]], ssem, add=True)
        pltpu.make_async_copy(buf, acc_seg.at[dst_v.at[j]], ssem).wait()
        pltpu.make_async_copy(dis_v.at[j], acc_dis.at[dst_v.at[j]], ssem).wait()
        pltpu.make_async_copy(ones_v, acc_cnt.at[dst_v.at[j]], ssem).wait()

    _gather_start(0, bufs[0])

    def _step(j0, _):
        for b in range(2):
            j = 2 * j0 + b
            buf = bufs[b]
            _gather_wait(j, buf)

            @pl.when(j + 1 < CHUNKS)
            def _():
                _gather_start(j + 1, bufs[1 - b])
            _scatter(j, buf)
        return 0
    lax.fori_loop(0, CHUNKS // 2, _step, 0)

    plsc.subcore_barrier()

    o0 = c * N_PAD + r0
    pltpu.sync_copy(acc_seg.at[pl.ds(r0, ROWS_PER_TILE)],
                    seg_out.at[pl.ds(o0, ROWS_PER_TILE)])
    pltpu.sync_copy(acc_cnt.at[pl.ds(r0, ROWS_PER_TILE)],
                    cnt_out.at[pl.ds(o0, ROWS_PER_TILE)])
    pltpu.sync_copy(acc_dis.at[pl.ds(r0, ROWS_PER_TILE)],
                    dis_out.at[pl.ds(o0, ROWS_PER_TILE)])


def _segment_sums(tbl, src2d, dst2d, dis2d):
    mesh = plsc.VectorSubcoreMesh(core_axis_name="c", subcore_axis_name="s",
                                  num_cores=NC, num_subcores=NS)
    f32 = jnp.float32
    return pl.kernel(
        _seg_body,
        out_type=[
            jax.ShapeDtypeStruct((NC * N_PAD, HID), f32),
            jax.ShapeDtypeStruct((NC * N_PAD,), f32),
            jax.ShapeDtypeStruct((NC * N_PAD,), f32),
        ],
        mesh=mesh,
        scratch_types=[
            pltpu.VMEM((CHUNKS, CH), jnp.int32),
            pltpu.VMEM((CHUNKS, CH), jnp.int32),
            pltpu.VMEM((CHUNKS, CH), f32),
            pltpu.VMEM((CH,), f32),
            pltpu.VMEM((CH, HID), f32),
            pltpu.VMEM((CH, HID), f32),
            pltpu.VMEM_SHARED((N_PAD, HID), f32),
            pltpu.VMEM_SHARED((N_PAD,), f32),
            pltpu.VMEM_SHARED((N_PAD,), f32),
            pltpu.SemaphoreType.DMA,
            pltpu.SemaphoreType.DMA,
        ],
        compiler_params=pltpu.CompilerParams(use_tc_tiling_on_sc=False),
    )(tbl, src2d, dst2d, dis2d)


def _final_body(pos, h, x, seg_a, seg_s, cns,
                wsp_a, bu, wdis_a, wsp_s, bx, wdis_s,
                w1, w2, w3, w4, w5, bd, out):
    dot = lambda a, b: jnp.dot(a[:], b[:], preferred_element_type=jnp.float32)
    cnt_a = cns[:, 0:1]
    sd_a = cns[:, 1:2]
    cnt_s = cns[:, 2:3]
    sd_s = cns[:, 3:4]
    su = seg_a[:] + cnt_a * (dot(pos, wsp_a) + bu[:]) + sd_a * wdis_a[:]
    mx = seg_s[:] + cnt_s * (dot(pos, wsp_s) + bx[:]) + sd_s * wdis_s[:]
    mx = mx / jnp.maximum(cnt_s, 1.0)
    out[:] = (dot(pos, w1) + dot(h, w2) + dot(su, w3) + dot(mx, w4)
              + dot(x, w5) + bd[:])


def _final(pos, h, x, seg_a, seg_s, cns,
           wsp_a, bu, wdis_a, wsp_s, bx, wdis_s, w1, w2, w3, w4, w5, bd):
    row = lambda k: pl.BlockSpec((BLK, k), lambda i: (i, 0))
    full = lambda a: pl.BlockSpec(a.shape, lambda i: (0, 0))
    return pl.pallas_call(
        _final_body,
        grid=(N // BLK,),
        in_specs=[row(2), row(HID), row(128), row(HID), row(HID), row(4),
                  full(wsp_a), full(bu), full(wdis_a),
                  full(wsp_s), full(bx), full(wdis_s),
                  full(w1), full(w2), full(w3), full(w4), full(w5), full(bd)],
        out_specs=pl.BlockSpec((BLK, HID), lambda i: (i, 0)),
        out_shape=jax.ShapeDtypeStruct((N, HID), jnp.float32),
    )(pos, h, x, seg_a, seg_s, cns,
      wsp_a, bu, wdis_a, wsp_s, bx, wdis_s, w1, w2, w3, w4, w5, bd)


def _pad_edges(a, fill):
    return jnp.concatenate([a, jnp.full((E_PAD - E,), fill, a.dtype)])


def kernel(h, x, u, state_pos, action_pos, a2s_edge_index, a2s_dis,
           s2s_edge_index, s2s_dis, W_u2h, b_u2h, W_x2h, b_x2h, W_upd, b_upd):
    f32 = jnp.float32

    a_in = jnp.concatenate([u, action_pos], axis=1)
    s_in = jnp.concatenate([x, h, state_pos], axis=1)
    tbl_a = _row_matmul(a_in, W_u2h[:, :18].T)
    tbl_s = _row_matmul(s_in, W_x2h[:, :194].T)
    tbl = jnp.concatenate([tbl_a, tbl_s], axis=0)

    src2d = jnp.concatenate([
        _pad_edges(a2s_edge_index[0], 0),
        _pad_edges(s2s_edge_index[0], 0) + N,
    ]).reshape(2 * EROWS, CH)
    dst2d = jnp.concatenate([
        _pad_edges(a2s_edge_index[1], N),
        _pad_edges(s2s_edge_index[1], N),
    ]).reshape(2 * EROWS, CH)
    dis2d = jnp.concatenate([
        _pad_edges(a2s_dis[:, 0], 0.0),
        _pad_edges(s2s_dis[:, 0], 0.0),
    ]).reshape(2 * EROWS, CH)

    seg, cnt, sdis = _segment_sums(tbl, src2d, dst2d, dis2d)

    seg_a, seg_s = seg[:N], seg[N_PAD:N_PAD + N]
    cns = jnp.stack([cnt[:N], sdis[:N], cnt[N_PAD:N_PAD + N],
                     sdis[N_PAD:N_PAD + N]], axis=1)

    out = _final(
        state_pos, h, x, seg_a, seg_s, cns,
        W_u2h[:, 18:20].T, b_u2h[None, :], W_u2h[:, 20][None, :],
        W_x2h[:, 194:196].T, b_x2h[None, :], W_x2h[:, 196][None, :],
        W_upd[:, 0:2].T, W_upd[:, 2:66].T, W_upd[:, 66:130].T,
        W_upd[:, 130:194].T, W_upd[:, 194:322].T, b_upd[None, :])
    return out


# speedup vs baseline: 13.3411x; 1.0856x over previous
import jax
import jax.numpy as jnp
from jax import lax
from jax.experimental import pallas as pl
from jax.experimental.pallas import tpu as pltpu
from jax.experimental.pallas import tpu_sc as plsc

N = 10000
HID = 64
E = 320000
NC, NS, L = 2, 16, 16
CH = 640
CHUNKS = 32
E_PAD = NS * CH * CHUNKS
ROWS_PER_TILE = 640
N_PAD = NS * ROWS_PER_TILE


BLK = 2000


def _mm_body(a, w, out):
    out[:] = jnp.dot(a[:], w[:], preferred_element_type=jnp.float32)


def _row_matmul(a, w):
    n, k = a.shape
    return pl.pallas_call(
        _mm_body,
        grid=(n // BLK,),
        in_specs=[
            pl.BlockSpec((BLK, k), lambda i: (i, 0)),
            pl.BlockSpec((k, HID), lambda i: (0, 0)),
        ],
        out_specs=pl.BlockSpec((BLK, HID), lambda i: (i, 0)),
        out_shape=jax.ShapeDtypeStruct((n, HID), jnp.float32),
    )(a, w)


def _seg_body(tbl, src, dst, dis, seg_out, cnt_out, dis_out,
              src_a, src_b, dst_a, dst_b, dis_a, dis_b, ones_v,
              rows_a, rows_b, acc_seg, acc_cnt, acc_dis, tsem, gsem, ssem):
    c = lax.axis_index("c")
    s = lax.axis_index("s")

    ones16 = jnp.ones((L,), jnp.float32)
    zeros16 = jnp.zeros((L,), jnp.float32)

    def _init_ones(i, _):
        ones_v[pl.ds(i * L, L)] = ones16
        return 0
    lax.fori_loop(0, CH // L, _init_ones, 0)

    def _zero_rows(i, _):
        rows_a[i, pl.ds(0, L)] = zeros16
        rows_a[i, pl.ds(L, L)] = zeros16
        rows_a[i, pl.ds(2 * L, L)] = zeros16
        rows_a[i, pl.ds(3 * L, L)] = zeros16
        return 0
    lax.fori_loop(0, ROWS_PER_TILE, _zero_rows, 0)

    def _zero_vec(i, _):
        dis_a[pl.ds(i * L, L)] = zeros16
        return 0
    lax.fori_loop(0, ROWS_PER_TILE // L, _zero_vec, 0)

    r0 = s * ROWS_PER_TILE
    pltpu.sync_copy(rows_a, acc_seg.at[pl.ds(r0, ROWS_PER_TILE)])
    pltpu.sync_copy(dis_a, acc_cnt.at[pl.ds(r0, ROWS_PER_TILE)])
    pltpu.sync_copy(dis_a, acc_dis.at[pl.ds(r0, ROWS_PER_TILE)])

    plsc.subcore_barrier()

    base0 = c * E_PAD + s * (CH * CHUNKS)
    bufs = ((src_a, dst_a, dis_a, rows_a), (src_b, dst_b, dis_b, rows_b))

    def _stage(j, b):
        sb, db, fb, _ = bufs[b]
        pltpu.async_copy(src.at[pl.ds(base0 + j * CH, CH)], sb, tsem)
        pltpu.async_copy(dst.at[pl.ds(base0 + j * CH, CH)], db, tsem)
        pltpu.async_copy(dis.at[pl.ds(base0 + j * CH, CH)], fb, tsem)

    def _stage_wait(j, b):
        sb, db, fb, _ = bufs[b]
        pltpu.make_async_copy(src.at[pl.ds(base0 + j * CH, CH)], sb, tsem).wait()
        pltpu.make_async_copy(dst.at[pl.ds(base0 + j * CH, CH)], db, tsem).wait()
        pltpu.make_async_copy(dis.at[pl.ds(base0 + j * CH, CH)], fb, tsem).wait()

    def _gather_start(b):
        sb, _, _, rb = bufs[b]
        pltpu.async_copy(tbl.at[sb], rb, gsem)

    def _gather_wait(b):
        sb, _, _, rb = bufs[b]
        pltpu.make_async_copy(tbl.at[sb], rb, gsem).wait()

    def _scatter(b):
        _, db, fb, rb = bufs[b]
        pltpu.async_copy(rb, acc_seg.at[db], ssem, add=True)
        pltpu.async_copy(fb, acc_dis.at[db], ssem, add=True)
        pltpu.async_copy(ones_v, acc_cnt.at[db], ssem, add=True)
        pltpu.make_async_copy(rb, acc_seg.at[db], ssem).wait()
        pltpu.make_async_copy(fb, acc_dis.at[db], ssem).wait()
        pltpu.make_async_copy(ones_v, acc_cnt.at[db], ssem).wait()

    _stage(0, 0)
    _stage_wait(0, 0)
    _gather_start(0)
    _stage(1, 1)

    def _step(j0, _):
        for b in range(2):
            j = 2 * j0 + b
            _gather_wait(b)

            @pl.when(j + 1 < CHUNKS)
            def _():
                _stage_wait(j + 1, 1 - b)
                _gather_start(1 - b)
            _scatter(b)

            @pl.when(j + 2 < CHUNKS)
            def _():
                _stage(j + 2, b)
        return 0
    lax.fori_loop(0, CHUNKS // 2, _step, 0)

    plsc.subcore_barrier()

    o0 = c * N_PAD + r0
    pltpu.sync_copy(acc_seg.at[pl.ds(r0, ROWS_PER_TILE)],
                    seg_out.at[pl.ds(o0, ROWS_PER_TILE)])
    pltpu.sync_copy(acc_cnt.at[pl.ds(r0, ROWS_PER_TILE)],
                    cnt_out.at[pl.ds(o0, ROWS_PER_TILE)])
    pltpu.sync_copy(acc_dis.at[pl.ds(r0, ROWS_PER_TILE)],
                    dis_out.at[pl.ds(o0, ROWS_PER_TILE)])


def _segment_sums(tbl, src1d, dst1d, dis1d):
    mesh = plsc.VectorSubcoreMesh(core_axis_name="c", subcore_axis_name="s",
                                  num_cores=NC, num_subcores=NS)
    f32 = jnp.float32
    i32 = jnp.int32
    return pl.kernel(
        _seg_body,
        out_type=[
            jax.ShapeDtypeStruct((NC * N_PAD, HID), f32),
            jax.ShapeDtypeStruct((NC * N_PAD,), f32),
            jax.ShapeDtypeStruct((NC * N_PAD,), f32),
        ],
        mesh=mesh,
        scratch_types=[
            pltpu.VMEM((CH,), i32),
            pltpu.VMEM((CH,), i32),
            pltpu.VMEM((CH,), i32),
            pltpu.VMEM((CH,), i32),
            pltpu.VMEM((CH,), f32),
            pltpu.VMEM((CH,), f32),
            pltpu.VMEM((CH,), f32),
            pltpu.VMEM((CH, HID), f32),
            pltpu.VMEM((CH, HID), f32),
            pltpu.VMEM_SHARED((N_PAD, HID), f32),
            pltpu.VMEM_SHARED((N_PAD,), f32),
            pltpu.VMEM_SHARED((N_PAD,), f32),
            pltpu.SemaphoreType.DMA,
            pltpu.SemaphoreType.DMA,
            pltpu.SemaphoreType.DMA,
        ],
        compiler_params=pltpu.CompilerParams(use_tc_tiling_on_sc=False),
    )(tbl, src1d, dst1d, dis1d)


def _final_body(pos, h, x, seg_a, seg_s, cns,
                wsp_a, bu, wdis_a, wsp_s, bx, wdis_s,
                w1, w2, w3, w4, w5, bd, out):
    dot = lambda a, b: jnp.dot(a[:], b[:], preferred_element_type=jnp.float32)
    cnt_a = cns[:, 0:1]
    sd_a = cns[:, 1:2]
    cnt_s = cns[:, 2:3]
    sd_s = cns[:, 3:4]
    su = seg_a[:] + cnt_a * (dot(pos, wsp_a) + bu[:]) + sd_a * wdis_a[:]
    mx = seg_s[:] + cnt_s * (dot(pos, wsp_s) + bx[:]) + sd_s * wdis_s[:]
    mx = mx / jnp.maximum(cnt_s, 1.0)
    out[:] = (dot(pos, w1) + dot(h, w2) + dot(su, w3) + dot(mx, w4)
              + dot(x, w5) + bd[:])


def _final(pos, h, x, seg_a, seg_s, cns,
           wsp_a, bu, wdis_a, wsp_s, bx, wdis_s, w1, w2, w3, w4, w5, bd):
    row = lambda k: pl.BlockSpec((BLK, k), lambda i: (i, 0))
    full = lambda a: pl.BlockSpec(a.shape, lambda i: (0, 0))
    return pl.pallas_call(
        _final_body,
        grid=(N // BLK,),
        in_specs=[row(2), row(HID), row(128), row(HID), row(HID), row(4),
                  full(wsp_a), full(bu), full(wdis_a),
                  full(wsp_s), full(bx), full(wdis_s),
                  full(w1), full(w2), full(w3), full(w4), full(w5), full(bd)],
        out_specs=pl.BlockSpec((BLK, HID), lambda i: (i, 0)),
        out_shape=jax.ShapeDtypeStruct((N, HID), jnp.float32),
    )(pos, h, x, seg_a, seg_s, cns,
      wsp_a, bu, wdis_a, wsp_s, bx, wdis_s, w1, w2, w3, w4, w5, bd)


def _pad_edges(a, fill):
    return jnp.concatenate([a, jnp.full((E_PAD - E,), fill, a.dtype)])


def kernel(h, x, u, state_pos, action_pos, a2s_edge_index, a2s_dis,
           s2s_edge_index, s2s_dis, W_u2h, b_u2h, W_x2h, b_x2h, W_upd, b_upd):
    f32 = jnp.float32

    a_in = jnp.concatenate([u, action_pos], axis=1)
    s_in = jnp.concatenate([x, h, state_pos], axis=1)
    tbl_a = _row_matmul(a_in, W_u2h[:, :18].T)
    tbl_s = _row_matmul(s_in, W_x2h[:, :194].T)
    tbl = jnp.concatenate([tbl_a, tbl_s], axis=0)

    src1d = jnp.concatenate([
        _pad_edges(a2s_edge_index[0], 0),
        _pad_edges(s2s_edge_index[0], 0) + N,
    ])
    dst1d = jnp.concatenate([
        _pad_edges(a2s_edge_index[1], N),
        _pad_edges(s2s_edge_index[1], N),
    ])
    dis1d = jnp.concatenate([
        _pad_edges(a2s_dis[:, 0], 0.0),
        _pad_edges(s2s_dis[:, 0], 0.0),
    ])

    seg, cnt, sdis = _segment_sums(tbl, src1d, dst1d, dis1d)

    seg_a, seg_s = seg[:N], seg[N_PAD:N_PAD + N]
    cns = jnp.stack([cnt[:N], sdis[:N], cnt[N_PAD:N_PAD + N],
                     sdis[N_PAD:N_PAD + N]], axis=1)

    out = _final(
        state_pos, h, x, seg_a, seg_s, cns,
        W_u2h[:, 18:20].T, b_u2h[None, :], W_u2h[:, 20][None, :],
        W_x2h[:, 194:196].T, b_x2h[None, :], W_x2h[:, 196][None, :],
        W_upd[:, 0:2].T, W_upd[:, 2:66].T, W_upd[:, 66:130].T,
        W_upd[:, 130:194].T, W_upd[:, 194:322].T, b_upd[None, :])
    return out


# speedup vs baseline: 25.2802x; 1.8949x over previous
import jax
import jax.numpy as jnp
from jax import lax
from jax.experimental import pallas as pl
from jax.experimental.pallas import tpu as pltpu
from jax.experimental.pallas import tpu_sc as plsc

N = 10000
HID = 64
E = 320000
NC, NS, L = 2, 16, 16
EPT = E // NS
CH = 640
FULL = EPT // CH
TAIL = EPT - FULL * CH
ROWS_PER_TILE = 640
N_PAD = NS * ROWS_PER_TILE
BLK = 2000


def _tbl_body(*refs):
    ins, w, out = refs[:-1][0::2], refs[:-1][1::2], refs[-1]
    acc = jnp.dot(ins[0][:], w[0][:], preferred_element_type=jnp.float32)
    for r, wr in zip(ins[1:], w[1:]):
        acc += jnp.dot(r[:], wr[:], preferred_element_type=jnp.float32)
    out[:] = acc


def _table(pairs):
    operands = []
    specs = []
    for a, w in pairs:
        operands += [a, w]
        k = a.shape[1]
        specs += [pl.BlockSpec((BLK, k), lambda i: (i, 0)),
                  pl.BlockSpec((k, HID), lambda i: (0, 0))]
    return pl.pallas_call(
        _tbl_body,
        grid=(N // BLK,),
        in_specs=specs,
        out_specs=pl.BlockSpec((BLK, HID), lambda i: (i, 0)),
        out_shape=jax.ShapeDtypeStruct((N, HID), jnp.float32),
    )(*operands)


def _seg_body(tbl_a, tbl_s, a_ei, s_ei, a_dis, s_dis,
              seg_out, cnt_out, dis_out,
              src_a, src_b, dst_a, dst_b, dis_a, dis_b, ones_v,
              rows_a, rows_b, acc_seg, acc_cnt, acc_dis, tsem, gsem, ssem):
    c = lax.axis_index("c")
    s = lax.axis_index("s")

    ones16 = jnp.ones((L,), jnp.float32)
    zeros16 = jnp.zeros((L,), jnp.float32)

    def _init_ones(i, _):
        ones_v[pl.ds(i * L, L)] = ones16
        return 0
    lax.fori_loop(0, CH // L, _init_ones, 0)

    def _zero_rows(i, _):
        rows_a[i, pl.ds(0, L)] = zeros16
        rows_a[i, pl.ds(L, L)] = zeros16
        rows_a[i, pl.ds(2 * L, L)] = zeros16
        rows_a[i, pl.ds(3 * L, L)] = zeros16
        return 0
    lax.fori_loop(0, ROWS_PER_TILE, _zero_rows, 0)

    def _zero_vec(i, _):
        dis_a[pl.ds(i * L, L)] = zeros16
        return 0
    lax.fori_loop(0, ROWS_PER_TILE // L, _zero_vec, 0)

    r0 = s * ROWS_PER_TILE
    pltpu.sync_copy(rows_a, acc_seg.at[pl.ds(r0, ROWS_PER_TILE)])
    pltpu.sync_copy(dis_a, acc_cnt.at[pl.ds(r0, ROWS_PER_TILE)])
    pltpu.sync_copy(dis_a, acc_dis.at[pl.ds(r0, ROWS_PER_TILE)])

    plsc.subcore_barrier()

    def _bycore(fn):
        @pl.when(c == 0)
        def _():
            fn(a_ei, a_dis, tbl_a)

        @pl.when(c == 1)
        def _():
            fn(s_ei, s_dis, tbl_s)

    base0 = s * EPT
    bufs = ((src_a, dst_a, dis_a, rows_a), (src_b, dst_b, dis_b, rows_b))

    def _stage(j, b, n):
        sb, db, fb, _ = bufs[b]

        def fn(ei, dref, _tbl):
            pltpu.async_copy(ei.at[0, pl.ds(base0 + j * CH, n)],
                             sb.at[pl.ds(0, n)], tsem)
            pltpu.async_copy(ei.at[1, pl.ds(base0 + j * CH, n)],
                             db.at[pl.ds(0, n)], tsem)
            pltpu.async_copy(dref.at[pl.ds(base0 + j * CH, n)],
                             fb.at[pl.ds(0, n)], tsem)
        _bycore(fn)

    def _stage_wait(j, b, n):
        sb, db, fb, _ = bufs[b]

        def fn(ei, dref, _tbl):
            pltpu.make_async_copy(ei.at[0, pl.ds(base0 + j * CH, n)],
                                  sb.at[pl.ds(0, n)], tsem).wait()
            pltpu.make_async_copy(ei.at[1, pl.ds(base0 + j * CH, n)],
                                  db.at[pl.ds(0, n)], tsem).wait()
            pltpu.make_async_copy(dref.at[pl.ds(base0 + j * CH, n)],
                                  fb.at[pl.ds(0, n)], tsem).wait()
        _bycore(fn)

    def _gather_start(b, n):
        sb, _, _, rb = bufs[b]

        def fn(_ei, _dref, tbl):
            pltpu.async_copy(tbl.at[sb.at[pl.ds(0, n)]],
                             rb.at[pl.ds(0, n)], gsem)
        _bycore(fn)

    def _gather_wait(b, n):
        sb, _, _, rb = bufs[b]

        def fn(_ei, _dref, tbl):
            pltpu.make_async_copy(tbl.at[sb.at[pl.ds(0, n)]],
                                  rb.at[pl.ds(0, n)], gsem).wait()
        _bycore(fn)

    def _scatter(b, n):
        _, db, fb, rb = bufs[b]
        dbn = db.at[pl.ds(0, n)]
        pltpu.async_copy(rb.at[pl.ds(0, n)], acc_seg.at[dbn], ssem, add=True)
        pltpu.async_copy(fb.at[pl.ds(0, n)], acc_dis.at[dbn], ssem, add=True)
        pltpu.async_copy(ones_v.at[pl.ds(0, n)], acc_cnt.at[dbn], ssem,
                         add=True)
        pltpu.make_async_copy(rb.at[pl.ds(0, n)], acc_seg.at[dbn], ssem).wait()
        pltpu.make_async_copy(fb.at[pl.ds(0, n)], acc_dis.at[dbn], ssem).wait()
        pltpu.make_async_copy(ones_v.at[pl.ds(0, n)], acc_cnt.at[dbn],
                              ssem).wait()

    _stage(0, 0, CH)
    _stage_wait(0, 0, CH)
    _gather_start(0, CH)
    _stage(1, 1, CH)

    def _step(j0, _):
        for b in range(2):
            j = 2 * j0 + b

            @pl.when(j < FULL)
            def _():
                _gather_wait(b, CH)

                @pl.when(j + 1 < FULL)
                def _():
                    _stage_wait(j + 1, 1 - b, CH)
                    _gather_start(1 - b, CH)
                _scatter(b, CH)

                @pl.when(j + 2 < FULL)
                def _():
                    _stage(j + 2, b, CH)
        return 0
    lax.fori_loop(0, (FULL + 1) // 2, _step, 0)

    _stage(FULL, 0, TAIL)
    _stage_wait(FULL, 0, TAIL)
    _gather_start(0, TAIL)
    _gather_wait(0, TAIL)
    _scatter(0, TAIL)

    plsc.subcore_barrier()

    o0 = c * N + r0

    @pl.when(s < NS - 1)
    def _():
        pltpu.sync_copy(acc_seg.at[pl.ds(r0, ROWS_PER_TILE)],
                        seg_out.at[pl.ds(o0, ROWS_PER_TILE)])
        pltpu.sync_copy(acc_cnt.at[pl.ds(r0, ROWS_PER_TILE)],
                        cnt_out.at[pl.ds(o0, ROWS_PER_TILE)])
        pltpu.sync_copy(acc_dis.at[pl.ds(r0, ROWS_PER_TILE)],
                        dis_out.at[pl.ds(o0, ROWS_PER_TILE)])

    last = N - (NS - 1) * ROWS_PER_TILE

    @pl.when(s == NS - 1)
    def _():
        pltpu.sync_copy(acc_seg.at[pl.ds(r0, last)],
                        seg_out.at[pl.ds(o0, last)])
        pltpu.sync_copy(acc_cnt.at[pl.ds(r0, last)],
                        cnt_out.at[pl.ds(o0, last)])
        pltpu.sync_copy(acc_dis.at[pl.ds(r0, last)],
                        dis_out.at[pl.ds(o0, last)])


def _segment_sums(tbl_a, tbl_s, a_ei, s_ei, a_dis, s_dis):
    mesh = plsc.VectorSubcoreMesh(core_axis_name="c", subcore_axis_name="s",
                                  num_cores=NC, num_subcores=NS)
    f32 = jnp.float32
    i32 = jnp.int32
    return pl.kernel(
        _seg_body,
        out_type=[
            jax.ShapeDtypeStruct((NC * N, HID), f32),
            jax.ShapeDtypeStruct((NC * N,), f32),
            jax.ShapeDtypeStruct((NC * N,), f32),
        ],
        mesh=mesh,
        scratch_types=[
            pltpu.VMEM((CH,), i32),
            pltpu.VMEM((CH,), i32),
            pltpu.VMEM((CH,), i32),
            pltpu.VMEM((CH,), i32),
            pltpu.VMEM((CH,), f32),
            pltpu.VMEM((CH,), f32),
            pltpu.VMEM((CH,), f32),
            pltpu.VMEM((CH, HID), f32),
            pltpu.VMEM((CH, HID), f32),
            pltpu.VMEM_SHARED((N_PAD, HID), f32),
            pltpu.VMEM_SHARED((N_PAD,), f32),
            pltpu.VMEM_SHARED((N_PAD,), f32),
            pltpu.SemaphoreType.DMA,
            pltpu.SemaphoreType.DMA,
            pltpu.SemaphoreType.DMA,
        ],
        compiler_params=pltpu.CompilerParams(use_tc_tiling_on_sc=False),
    )(tbl_a, tbl_s, a_ei, s_ei, a_dis, s_dis)


def _final_body(pos, h, x, seg_a, seg_s, cnt_a, cnt_s, sd_a, sd_s,
                wsp_a, bu, wdis_a, wsp_s, bx, wdis_s,
                w1, w2, w3, w4, w5, bd, out):
    dot = lambda a, b: jnp.dot(a[:], b[:], preferred_element_type=jnp.float32)
    su = seg_a[:] + cnt_a[:] * (dot(pos, wsp_a) + bu[:]) + sd_a[:] * wdis_a[:]
    mx = seg_s[:] + cnt_s[:] * (dot(pos, wsp_s) + bx[:]) + sd_s[:] * wdis_s[:]
    mx = mx / jnp.maximum(cnt_s[:], 1.0)
    out[:] = (dot(pos, w1) + dot(h, w2) + dot(su, w3) + dot(mx, w4)
              + dot(x, w5) + bd[:])


def _final(pos, h, x, seg, cnt, sdis,
           wsp_a, bu, wdis_a, wsp_s, bx, wdis_s, w1, w2, w3, w4, w5, bd):
    nb = N // BLK
    rowa = lambda k: pl.BlockSpec((BLK, k), lambda i: (i, 0))
    rows = lambda k: pl.BlockSpec((BLK, k), lambda i: (i + nb, 0))
    full = lambda a: pl.BlockSpec(a.shape, lambda i: (0, 0))
    return pl.pallas_call(
        _final_body,
        grid=(nb,),
        in_specs=[rowa(2), rowa(HID), rowa(128),
                  rowa(HID), rows(HID), rowa(1), rows(1), rowa(1), rows(1),
                  full(wsp_a), full(bu), full(wdis_a),
                  full(wsp_s), full(bx), full(wdis_s),
                  full(w1), full(w2), full(w3), full(w4), full(w5), full(bd)],
        out_specs=pl.BlockSpec((BLK, HID), lambda i: (i, 0)),
        out_shape=jax.ShapeDtypeStruct((N, HID), jnp.float32),
    )(pos, h, x, seg, seg, cnt, cnt, sdis, sdis,
      wsp_a, bu, wdis_a, wsp_s, bx, wdis_s, w1, w2, w3, w4, w5, bd)


def kernel(h, x, u, state_pos, action_pos, a2s_edge_index, a2s_dis,
           s2s_edge_index, s2s_dis, W_u2h, b_u2h, W_x2h, b_x2h, W_upd, b_upd):
    tbl_a = _table([(u, W_u2h[:, :16].T), (action_pos, W_u2h[:, 16:18].T)])
    tbl_s = _table([(x, W_x2h[:, :128].T), (h, W_x2h[:, 128:192].T),
                    (state_pos, W_x2h[:, 192:194].T)])

    seg, cnt, sdis = _segment_sums(
        tbl_a, tbl_s, a2s_edge_index, s2s_edge_index,
        a2s_dis.reshape(E), s2s_dis.reshape(E))

    out = _final(
        state_pos, h, x, seg, cnt.reshape(NC * N, 1), sdis.reshape(NC * N, 1),
        W_u2h[:, 18:20].T, b_u2h[None, :], W_u2h[:, 20][None, :],
        W_x2h[:, 194:196].T, b_x2h[None, :], W_x2h[:, 196][None, :],
        W_upd[:, 0:2].T, W_upd[:, 2:66].T, W_upd[:, 66:130].T,
        W_upd[:, 130:194].T, W_upd[:, 194:322].T, b_upd[None, :])
    return out
